# baseline (speedup 1.0000x reference)
import functools

import jax
import jax.numpy as jnp
from jax import lax
from jax.experimental import pallas as pl
from jax.experimental.pallas import tpu as pltpu
from jax.experimental.pallas import tpu_sc as plsc

D = 128
NC = 2
NS = 16
CHUNK = 128
RPT = 640
ACC_ROWS = NS * RPT
NB = 2
G = 20
NG = 8


def _sc_scatter_add(feat, src5, dst5, zeros):
    mesh = plsc.VectorSubcoreMesh(core_axis_name="c", subcore_axis_name="s")

    @functools.partial(
        pl.kernel,
        mesh=mesh,
        out_type=jax.ShapeDtypeStruct((ACC_ROWS, D), jnp.float32),
        scratch_types=[
            *[pltpu.VMEM((2, G, CHUNK), jnp.int32) for _ in range(2)],
            *[pltpu.VMEM((CHUNK, D), jnp.float32) for _ in range(NB)],
            pltpu.VMEM_SHARED((ACC_ROWS, D), jnp.float32),
            *[pltpu.SemaphoreType.DMA for _ in range(2 * NB + 2)],
        ],
    )
    def k(feat_h, src_h, dst_h, zeros_h, out_h, ib0, ib1, *rest):
        ibufs = (ib0, ib1)
        bufs = rest[:NB]
        acc_s = rest[NB]
        gsem = rest[NB + 1: 2 * NB + 1]
        ssem = rest[2 * NB + 1: 3 * NB + 1]
        isem = rest[3 * NB + 1: 3 * NB + 3]
        c = lax.axis_index("c")
        s = lax.axis_index("s")

        def idx_start(grp, par):
            pltpu.async_copy(src_h.at[s, grp], ibufs[par].at[0], isem[par])
            pltpu.async_copy(dst_h.at[s, grp], ibufs[par].at[1], isem[par])

        def idx_wait(grp, par):
            pltpu.make_async_copy(
                src_h.at[s, grp], ibufs[par].at[0], isem[par]
            ).wait()
            pltpu.make_async_copy(
                dst_h.at[s, grp], ibufs[par].at[1], isem[par]
            ).wait()

        @pl.when(c == 0)
        def _core0():
            zcopy = pltpu.async_copy(
                zeros_h.at[s], acc_s.at[pl.ds(s * RPT, RPT)], isem[1]
            )
            idx_start(0, 0)
            idx_wait(0, 0)
            zcopy.wait()
            plsc.subcore_barrier()

            for grp in range(NG):
                ib = ibufs[grp % 2]

                def g_start(t, b):
                    pltpu.async_copy(feat_h.at[ib.at[0, t]], bufs[b], gsem[b])

                def g_wait(t, b):
                    pltpu.make_async_copy(
                        feat_h.at[ib.at[0, t]], bufs[b], gsem[b]
                    ).wait()

                def s_start(t, b):
                    pltpu.async_copy(
                        bufs[b], acc_s.at[ib.at[1, t]], ssem[b], add=True
                    )

                def s_wait(t, b):
                    pltpu.make_async_copy(
                        bufs[b], acc_s.at[ib.at[1, t]], ssem[b]
                    ).wait()

                if grp > 0:
                    idx_wait(grp, grp % 2)
                if grp + 1 < NG:
                    idx_start(grp + 1, (grp + 1) % 2)
                g_start(0, 0)
                g_start(1, 1)
                g_wait(0, 0)
                s_start(0, 0)
                g_wait(1, 1)
                s_start(1, 1)
                s_wait(0, 0)
                g_start(2, 0)

                def body(i, _):
                    j0 = 2 * i
                    j1 = j0 + 1
                    g_wait(j0, 0)
                    s_start(j0, 0)
                    s_wait(j1, 1)
                    g_start(j0 + 1, 1)
                    g_wait(j1, 1)
                    s_start(j1, 1)
                    s_wait(j0, 0)
                    g_start(j1 + 1, 0)
                    return ()

                lax.fori_loop(1, G // 2 - 1, body, ())
                g_wait(G - 2, 0)
                s_start(G - 2, 0)
                s_wait(G - 3, 1)
                g_start(G - 1, 1)
                g_wait(G - 1, 1)
                s_start(G - 1, 1)
                s_wait(G - 2, 0)
                s_wait(G - 1, 1)

            plsc.subcore_barrier()
            pltpu.sync_copy(
                acc_s.at[pl.ds(s * RPT, RPT)], out_h.at[pl.ds(s * RPT, RPT)]
            )

    return k(feat, src5, dst5, zeros)


def _tc_linear(acc, W, b, n_nodes):
    blk = 1000
    grid = n_nodes // blk

    def body(p_ref, w_ref, b_ref, o_ref):
        y = lax.dot_general(
            p_ref[...], w_ref[...], (((1,), (1,)), ((), ())),
            preferred_element_type=jnp.float32,
        )
        o_ref[...] = y + b_ref[...]

    return pl.pallas_call(
        body,
        grid=(grid,),
        in_specs=[
            pl.BlockSpec((blk, D), lambda i: (i, 0)),
            pl.BlockSpec((D, D), lambda i: (0, 0)),
            pl.BlockSpec((1, D), lambda i: (0, 0)),
        ],
        out_specs=pl.BlockSpec((blk, D), lambda i: (i, 0)),
        out_shape=jax.ShapeDtypeStruct((n_nodes, D), jnp.float32),
    )(acc, W, b.reshape(1, D))


def kernel(feat, edge_index, W, b):
    n_nodes = feat.shape[0]
    n_edges = edge_index.shape[1]
    src = edge_index[0].astype(jnp.int32)
    dst = edge_index[1].astype(jnp.int32)
    e_pad = NS * NG * G * CHUNK
    pad = e_pad - n_edges
    if pad:
        ar = jnp.arange(pad, dtype=jnp.int32)
        src = jnp.concatenate([src, ar % n_nodes])
        dst = jnp.concatenate([dst, n_nodes + ar % (ACC_ROWS - n_nodes)])
    s5 = src.reshape(NS, NG, G, CHUNK)
    d5 = dst.reshape(NS, NG, G, CHUNK)
    zeros = jnp.zeros((NS, RPT, D), jnp.float32)
    acc = _sc_scatter_add(feat, s5, d5, zeros)
    return _tc_linear(acc, W, b, n_nodes)


# speedup vs baseline: 1.1784x; 1.1784x over previous
import functools

import jax
import jax.numpy as jnp
from jax import lax
from jax.experimental import pallas as pl
from jax.experimental.pallas import tpu as pltpu
from jax.experimental.pallas import tpu_sc as plsc

D = 128
NC = 2
NS = 16
CHUNK = 128
RPT = 640
ACC_ROWS = NS * RPT
NB = 2
G = 20
NG = 8


def _sc_scatter_add(feat, src5, dst5, zeros):
    mesh = plsc.VectorSubcoreMesh(core_axis_name="c", subcore_axis_name="s")

    @functools.partial(
        pl.kernel,
        mesh=mesh,
        out_type=jax.ShapeDtypeStruct((ACC_ROWS, D), jnp.float32),
        scratch_types=[
            *[pltpu.VMEM((2, G, CHUNK), jnp.int32) for _ in range(2)],
            *[pltpu.VMEM((CHUNK, D), jnp.float32) for _ in range(NB)],
            pltpu.VMEM_SHARED((ACC_ROWS, D), jnp.float32),
            *[pltpu.SemaphoreType.DMA for _ in range(NB + 3)],
        ],
    )
    def k(feat_h, src_h, dst_h, zeros_h, out_h, ib0, ib1, *rest):
        ibufs = (ib0, ib1)
        bufs = rest[:NB]
        acc_s = rest[NB]
        gsem = rest[NB + 1: 2 * NB + 1]
        isem = rest[2 * NB + 1: 2 * NB + 3]
        c = lax.axis_index("c")
        s = lax.axis_index("s")

        def idx_start(grp, par):
            pltpu.async_copy(src_h.at[s, grp], ibufs[par].at[0], isem[par])
            pltpu.async_copy(dst_h.at[s, grp], ibufs[par].at[1], isem[par])

        def idx_wait(grp, par):
            pltpu.make_async_copy(
                src_h.at[s, grp], ibufs[par].at[0], isem[par]
            ).wait()
            pltpu.make_async_copy(
                dst_h.at[s, grp], ibufs[par].at[1], isem[par]
            ).wait()

        @pl.when(c == 0)
        def _core0():
            zcopy = pltpu.async_copy(
                zeros_h.at[s], acc_s.at[pl.ds(s * RPT, RPT)], isem[1]
            )
            idx_start(0, 0)
            idx_wait(0, 0)
            zcopy.wait()
            plsc.subcore_barrier()

            for grp in range(NG):
                ib = ibufs[grp % 2]
                if grp > 0:
                    idx_wait(grp, grp % 2)
                if grp + 1 < NG:
                    idx_start(grp + 1, (grp + 1) % 2)
                for b in range(NB):
                    pltpu.async_copy(feat_h.at[ib.at[0, b]], bufs[b], gsem[b])

                def body(i, _):
                    for b in range(NB):
                        t = i * NB + b
                        pltpu.make_async_copy(
                            feat_h.at[ib.at[0, t]], bufs[b], gsem[b]
                        ).wait()
                        pltpu.sync_copy(bufs[b], acc_s.at[ib.at[1, t]],
                                        add=True)
                        pltpu.async_copy(
                            feat_h.at[ib.at[0, t + NB]], bufs[b], gsem[b]
                        )
                    return ()

                lax.fori_loop(0, (G - NB) // NB, body, ())
                for b in range(NB):
                    t = G - NB + b
                    pltpu.make_async_copy(
                        feat_h.at[ib.at[0, t]], bufs[b], gsem[b]
                    ).wait()
                    pltpu.sync_copy(bufs[b], acc_s.at[ib.at[1, t]], add=True)

            plsc.subcore_barrier()
            pltpu.sync_copy(
                acc_s.at[pl.ds(s * RPT, RPT)], out_h.at[pl.ds(s * RPT, RPT)]
            )

    return k(feat, src5, dst5, zeros)


def _tc_linear(acc, W, b, n_nodes):
    blk = 1000
    grid = n_nodes // blk

    def body(p_ref, w_ref, b_ref, o_ref):
        y = lax.dot_general(
            p_ref[...], w_ref[...], (((1,), (1,)), ((), ())),
            preferred_element_type=jnp.float32,
        )
        o_ref[...] = y + b_ref[...]

    return pl.pallas_call(
        body,
        grid=(grid,),
        in_specs=[
            pl.BlockSpec((blk, D), lambda i: (i, 0)),
            pl.BlockSpec((D, D), lambda i: (0, 0)),
            pl.BlockSpec((1, D), lambda i: (0, 0)),
        ],
        out_specs=pl.BlockSpec((blk, D), lambda i: (i, 0)),
        out_shape=jax.ShapeDtypeStruct((n_nodes, D), jnp.float32),
    )(acc, W, b.reshape(1, D))


def kernel(feat, edge_index, W, b):
    n_nodes = feat.shape[0]
    n_edges = edge_index.shape[1]
    src = edge_index[0].astype(jnp.int32)
    dst = edge_index[1].astype(jnp.int32)
    e_pad = NS * NG * G * CHUNK
    pad = e_pad - n_edges
    if pad:
        ar = jnp.arange(pad, dtype=jnp.int32)
        src = jnp.concatenate([src, ar % n_nodes])
        dst = jnp.concatenate([dst, n_nodes + ar % (ACC_ROWS - n_nodes)])
    s5 = src.reshape(NS, NG, G, CHUNK)
    d5 = dst.reshape(NS, NG, G, CHUNK)
    zeros = jnp.zeros((NS, RPT, D), jnp.float32)
    acc = _sc_scatter_add(feat, s5, d5, zeros)
    return _tc_linear(acc, W, b, n_nodes)


# speedup vs baseline: 1.2235x; 1.0383x over previous
import functools

import jax
import jax.numpy as jnp
from jax import lax
from jax.experimental import pallas as pl
from jax.experimental.pallas import tpu as pltpu
from jax.experimental.pallas import tpu_sc as plsc

D = 128
NC = 2
NS = 16
CHUNK = 64
RPT = 640
ACC_ROWS = NS * RPT
NB = 3
G = 36
NG = 9


def _sc_scatter_add(feat, src5, dst5, zeros):
    mesh = plsc.VectorSubcoreMesh(core_axis_name="c", subcore_axis_name="s")

    @functools.partial(
        pl.kernel,
        mesh=mesh,
        out_type=jax.ShapeDtypeStruct((ACC_ROWS, D), jnp.float32),
        scratch_types=[
            *[pltpu.VMEM((2, G, CHUNK), jnp.int32) for _ in range(2)],
            *[pltpu.VMEM((CHUNK, D), jnp.float32) for _ in range(NB)],
            pltpu.VMEM_SHARED((ACC_ROWS, D), jnp.float32),
            *[pltpu.SemaphoreType.DMA for _ in range(NB + 3)],
        ],
    )
    def k(feat_h, src_h, dst_h, zeros_h, out_h, ib0, ib1, *rest):
        ibufs = (ib0, ib1)
        bufs = rest[:NB]
        acc_s = rest[NB]
        gsem = rest[NB + 1: 2 * NB + 1]
        isem = rest[2 * NB + 1: 2 * NB + 3]
        c = lax.axis_index("c")
        s = lax.axis_index("s")

        def idx_start(grp, par):
            pltpu.async_copy(src_h.at[s, grp], ibufs[par].at[0], isem[par])
            pltpu.async_copy(dst_h.at[s, grp], ibufs[par].at[1], isem[par])

        def idx_wait(grp, par):
            pltpu.make_async_copy(
                src_h.at[s, grp], ibufs[par].at[0], isem[par]
            ).wait()
            pltpu.make_async_copy(
                dst_h.at[s, grp], ibufs[par].at[1], isem[par]
            ).wait()

        @pl.when(c == 0)
        def _core0():
            zcopy = pltpu.async_copy(
                zeros_h.at[s], acc_s.at[pl.ds(s * RPT, RPT)], isem[1]
            )
            idx_start(0, 0)
            idx_wait(0, 0)
            zcopy.wait()
            plsc.subcore_barrier()

            for grp in range(NG):
                ib = ibufs[grp % 2]
                if grp > 0:
                    idx_wait(grp, grp % 2)
                if grp + 1 < NG:
                    idx_start(grp + 1, (grp + 1) % 2)
                for b in range(NB):
                    pltpu.async_copy(feat_h.at[ib.at[0, b]], bufs[b], gsem[b])

                def body(i, _):
                    for b in range(NB):
                        t = i * NB + b
                        pltpu.make_async_copy(
                            feat_h.at[ib.at[0, t]], bufs[b], gsem[b]
                        ).wait()
                        pltpu.sync_copy(bufs[b], acc_s.at[ib.at[1, t]],
                                        add=True)
                        pltpu.async_copy(
                            feat_h.at[ib.at[0, t + NB]], bufs[b], gsem[b]
                        )
                    return ()

                lax.fori_loop(0, (G - NB) // NB, body, ())
                for b in range(NB):
                    t = G - NB + b
                    pltpu.make_async_copy(
                        feat_h.at[ib.at[0, t]], bufs[b], gsem[b]
                    ).wait()
                    pltpu.sync_copy(bufs[b], acc_s.at[ib.at[1, t]], add=True)

            plsc.subcore_barrier()
            pltpu.sync_copy(
                acc_s.at[pl.ds(s * RPT, RPT)], out_h.at[pl.ds(s * RPT, RPT)]
            )

    return k(feat, src5, dst5, zeros)


def _tc_linear(acc, W, b, n_nodes):
    blk = 1000
    grid = n_nodes // blk

    def body(p_ref, w_ref, b_ref, o_ref):
        y = lax.dot_general(
            p_ref[...], w_ref[...], (((1,), (1,)), ((), ())),
            preferred_element_type=jnp.float32,
        )
        o_ref[...] = y + b_ref[...]

    return pl.pallas_call(
        body,
        grid=(grid,),
        in_specs=[
            pl.BlockSpec((blk, D), lambda i: (i, 0)),
            pl.BlockSpec((D, D), lambda i: (0, 0)),
            pl.BlockSpec((1, D), lambda i: (0, 0)),
        ],
        out_specs=pl.BlockSpec((blk, D), lambda i: (i, 0)),
        out_shape=jax.ShapeDtypeStruct((n_nodes, D), jnp.float32),
    )(acc, W, b.reshape(1, D))


def kernel(feat, edge_index, W, b):
    n_nodes = feat.shape[0]
    n_edges = edge_index.shape[1]
    src = edge_index[0].astype(jnp.int32)
    dst = edge_index[1].astype(jnp.int32)
    e_pad = NS * NG * G * CHUNK
    pad = e_pad - n_edges
    if pad:
        ar = jnp.arange(pad, dtype=jnp.int32)
        src = jnp.concatenate([src, ar % n_nodes])
        dst = jnp.concatenate([dst, n_nodes + ar % (ACC_ROWS - n_nodes)])
    s5 = src.reshape(NS, NG, G, CHUNK)
    d5 = dst.reshape(NS, NG, G, CHUNK)
    zeros = jnp.zeros((NS, RPT, D), jnp.float32)
    acc = _sc_scatter_add(feat, s5, d5, zeros)
    return _tc_linear(acc, W, b, n_nodes)
